# baseline (speedup 1.0000x reference)
import functools

import jax
import jax.numpy as jnp
from jax import lax
from jax.experimental import pallas as pl
from jax.experimental.pallas import tpu as pltpu
from jax.experimental.pallas import tpu_sc as plsc

_B = 16384
_D = 128

_info = plsc.get_sparse_core_info()
_NC, _NS = _info.num_cores, _info.num_subcores
_NW = _NC * _NS
_BPW = _B // _NW
_CH = 128
_CPT = _BPW // _CH
_NCHUNK = 2 * _CPT
_NBUF = 4

_mesh = plsc.VectorSubcoreMesh(core_axis_name="c", subcore_axis_name="s")


@functools.partial(
    pl.kernel,
    out_type=(
        jax.ShapeDtypeStruct((_B, _D), jnp.float32),
        jax.ShapeDtypeStruct((_B, _D), jnp.float32),
    ),
    mesh=_mesh,
    scratch_types=(
        [pltpu.VMEM((_CPT, _CH), jnp.int32)] * 2
        + [pltpu.VMEM((_CH, _D), jnp.float32)] * _NBUF
        + [pltpu.SemaphoreType.DMA] * (2 * _NBUF)
    ),
)
def _gather2(X_hbm, y_hbm, W1_hbm, W2_hbm, out1_hbm, out2_hbm,
             idx1_v, idx2_v, *bufs_and_sems):
    rows = bufs_and_sems[:_NBUF]
    gsem = bufs_and_sems[_NBUF:2 * _NBUF]
    wsem = bufs_and_sems[2 * _NBUF:]

    wid = lax.axis_index("s") * _NC + lax.axis_index("c")
    base = wid * _BPW

    for c in range(_CPT):
        pltpu.sync_copy(X_hbm.at[pl.ds(base + c * _CH, _CH)], idx1_v.at[c])
        pltpu.sync_copy(y_hbm.at[pl.ds(base + c * _CH, _CH)], idx2_v.at[c])

    chunks = ([(W1_hbm, idx1_v, out1_hbm, c) for c in range(_CPT)]
              + [(W2_hbm, idx2_v, out2_hbm, c) for c in range(_CPT)])

    def fire_gather(i):
        tbl, idxv, _, c = chunks[i]
        b = i % _NBUF
        return pltpu.async_copy(tbl.at[idxv.at[c]], rows[b], gsem[b])

    gathers = {}
    writes = {}
    for i in range(_NBUF):
        gathers[i] = fire_gather(i)

    for i in range(_NCHUNK):
        b = i % _NBUF
        _, _, out, c = chunks[i]
        gathers[i].wait()
        writes[i] = pltpu.async_copy(
            rows[b], out.at[pl.ds(base + c * _CH, _CH)], wsem[b])
        if i + _NBUF < _NCHUNK:
            writes[i].wait()
            gathers[i + _NBUF] = fire_gather(i + _NBUF)

    for i in range(_NCHUNK - _NBUF, _NCHUNK):
        writes[i].wait()


def kernel(X, y, W1, W2):
    return _gather2(X, y, W1, W2)


# speedup vs baseline: 1.1141x; 1.1141x over previous
import functools

import jax
import jax.numpy as jnp
from jax import lax
from jax.experimental import pallas as pl
from jax.experimental.pallas import tpu as pltpu
from jax.experimental.pallas import tpu_sc as plsc

_B = 16384
_D = 128

_info = plsc.get_sparse_core_info()
_NC, _NS = _info.num_cores, _info.num_subcores
_NW = _NC * _NS
_BPW = _B // _NW
_CH = 128
_CPT = _BPW // _CH
_NCHUNK = 2 * _CPT
_NBUF = 7

_mesh = plsc.VectorSubcoreMesh(core_axis_name="c", subcore_axis_name="s")


@functools.partial(
    pl.kernel,
    out_type=(
        jax.ShapeDtypeStruct((_B, _D), jnp.float32),
        jax.ShapeDtypeStruct((_B, _D), jnp.float32),
    ),
    mesh=_mesh,
    scratch_types=(
        [pltpu.VMEM((_CPT, _CH), jnp.int32)] * 2
        + [pltpu.VMEM((_CH, _D), jnp.float32)] * _NBUF
        + [pltpu.SemaphoreType.DMA] * (2 * _NBUF)
    ),
)
def _gather2(X_hbm, y_hbm, W1_hbm, W2_hbm, out1_hbm, out2_hbm,
             idx1_v, idx2_v, *bufs_and_sems):
    rows = bufs_and_sems[:_NBUF]
    gsem = bufs_and_sems[_NBUF:2 * _NBUF]
    wsem = bufs_and_sems[2 * _NBUF:]

    wid = lax.axis_index("s") * _NC + lax.axis_index("c")
    base = wid * _BPW

    pltpu.sync_copy(X_hbm.at[wid], idx1_v)
    pltpu.sync_copy(y_hbm.at[wid], idx2_v)

    chunks = ([(W1_hbm, idx1_v, out1_hbm, c) for c in range(_CPT)]
              + [(W2_hbm, idx2_v, out2_hbm, c) for c in range(_CPT)])

    def fire_gather(i):
        tbl, idxv, _, c = chunks[i]
        b = i % _NBUF
        return pltpu.async_copy(tbl.at[idxv.at[c]], rows[b], gsem[b])

    gathers = {}
    writes = {}
    for i in range(min(_NBUF, _NCHUNK)):
        gathers[i] = fire_gather(i)

    for i in range(_NCHUNK):
        b = i % _NBUF
        _, _, out, c = chunks[i]
        gathers[i].wait()
        writes[i] = pltpu.async_copy(
            rows[b], out.at[pl.ds(base + c * _CH, _CH)], wsem[b])
        if i + _NBUF < _NCHUNK:
            writes[i].wait()
            gathers[i + _NBUF] = fire_gather(i + _NBUF)

    for i in range(max(0, _NCHUNK - _NBUF), _NCHUNK):
        writes[i].wait()


def kernel(X, y, W1, W2):
    Xr = X.reshape(_NW, _CPT, _CH)
    yr = y.reshape(_NW, _CPT, _CH)
    return _gather2(Xr, yr, W1, W2)
